# Initial kernel scaffold; baseline (speedup 1.0000x reference)
import jax
import jax.numpy as jnp
from jax.experimental import pallas as pl


def kernel(x, edge_index, edge_attr, edge_sh, W_in, W_sc, W1, b1, W2, b2, W_tp, W_pg, W_lo, W_out, sphere_dirs, sphere_Y):
    raise NotImplementedError("write your pallas kernel here")


# speedup vs baseline: 1.6789x; 1.6789x over previous
import functools
import math

import jax
import jax.numpy as jnp
from jax import lax
from jax.experimental import pallas as pl
from jax.experimental.pallas import tpu as pltpu
from jax.experimental.pallas import tpu_sc as plsc

_N = 10000
_E = 160000
_H = 240
_HP = 256
_NSH = 9
_NB = 6
_CUT = 0.35
_TN = 2000
_TE = 2000

_NC = 2
_NS = 16
_NW = _NC * _NS


def _sigmoid(v):
    return 1.0 / (1.0 + jnp.exp(-v))


def _node_in_body(x_ref, w_ref, o_ref):
    h = jnp.dot(x_ref[...], w_ref[...], preferred_element_type=jnp.float32)
    o_ref[...] = jnp.concatenate(
        [h, jnp.zeros((h.shape[0], _HP - _H), jnp.float32)], axis=1)


def _node_in(x, W_in):
    d = x.shape[1]
    return pl.pallas_call(
        _node_in_body,
        grid=(_N // _TN,),
        in_specs=[pl.BlockSpec((_TN, d), lambda i: (i, 0)),
                  pl.BlockSpec((d, _H), lambda i: (0, 0))],
        out_specs=pl.BlockSpec((_TN, _HP), lambda i: (i, 0)),
        out_shape=jax.ShapeDtypeStruct((_N, _HP), jnp.float32),
    )(x, W_in)


def _edge_body(ea_ref, sh_ref, xj_ref, w1_ref, b1_ref, w2_ref, b2_ref,
               wtp_ref, wpg_ref, m_ref):
    f32 = jnp.float32
    ea = ea_ref[...]
    centers = lax.broadcasted_iota(jnp.int32, (1, _NB), 1).astype(f32) * (
        _CUT / (_NB - 1))
    width = _CUT / _NB * 0.5
    diff = (ea - centers) * (1.0 / width)
    rbf = jnp.exp(-0.5 * diff * diff)
    cut = 0.5 * (jnp.cos(ea * (math.pi / _CUT)) + 1.0)
    cut = cut * (ea < _CUT).astype(f32)
    rbf = rbf * cut
    t = jnp.dot(rbf, w1_ref[...], preferred_element_type=f32) + b1_ref[...]
    t = t * _sigmoid(t)
    tpw = jnp.dot(t, w2_ref[...], preferred_element_type=f32) + b2_ref[...]
    a = jnp.dot(sh_ref[...], wtp_ref[...], preferred_element_type=f32)
    u = xj_ref[:, :_H] * a * tpw
    pre = jnp.dot(u, wpg_ref[...], preferred_element_type=f32)
    scal = pre[:, :64]
    scal = scal * _sigmoid(scal)
    gates = _sigmoid(pre[:, 64:112])
    gated = pre[:, 112:288]
    ji = lax.broadcasted_iota(jnp.int32, (48, 176), 1)
    ii = lax.broadcasted_iota(jnp.int32, (48, 176), 0)
    gidx = jnp.where(ji < 96, ji // 3, 32 + (ji - 96) // 5)
    sel = (gidx == ii).astype(f32)
    g = jnp.dot(gates, sel, preferred_element_type=f32)
    pad = jnp.zeros((scal.shape[0], _HP - _H), f32)
    m_ref[...] = jnp.concatenate([scal, gated * g, pad], axis=1)


def _edge_msg(edge_attr, edge_sh, xj, W1l, b1l, W2l, b2l, Wtpl, Wpgl):
    return pl.pallas_call(
        _edge_body,
        grid=(_E // _TE,),
        in_specs=[
            pl.BlockSpec((_TE, 1), lambda i: (i, 0)),
            pl.BlockSpec((_TE, _NSH), lambda i: (i, 0)),
            pl.BlockSpec((_TE, _HP), lambda i: (i, 0)),
            pl.BlockSpec((_NB, 32), lambda i: (0, 0)),
            pl.BlockSpec((1, 32), lambda i: (0, 0)),
            pl.BlockSpec((32, _H), lambda i: (0, 0)),
            pl.BlockSpec((1, _H), lambda i: (0, 0)),
            pl.BlockSpec((_NSH, _H), lambda i: (0, 0)),
            pl.BlockSpec((_H, 288), lambda i: (0, 0)),
        ],
        out_specs=pl.BlockSpec((_TE, _HP), lambda i: (i, 0)),
        out_shape=jax.ShapeDtypeStruct((_E, _HP), jnp.float32),
        compiler_params=pltpu.CompilerParams(
            dimension_semantics=("arbitrary",)),
    )(edge_attr, edge_sh, xj, W1l, b1l.reshape(1, -1), W2l,
      b2l.reshape(1, -1), Wtpl, Wpgl)


def _node_update_body(h_ref, agg_ref, wsc_ref, wlo_ref, o_ref):
    f32 = jnp.float32
    h2 = (jnp.dot(h_ref[:, :_H], wsc_ref[...], preferred_element_type=f32)
          + jnp.dot(agg_ref[:, :_H], wlo_ref[...], preferred_element_type=f32))
    o_ref[...] = jnp.concatenate(
        [h2, jnp.zeros((h2.shape[0], _HP - _H), f32)], axis=1)


def _node_update(h, agg, Wscl, Wlol):
    return pl.pallas_call(
        _node_update_body,
        grid=(_N // _TN,),
        in_specs=[pl.BlockSpec((_TN, _HP), lambda i: (i, 0)),
                  pl.BlockSpec((_TN, _HP), lambda i: (i, 0)),
                  pl.BlockSpec((_H, _H), lambda i: (0, 0)),
                  pl.BlockSpec((_H, _H), lambda i: (0, 0))],
        out_specs=pl.BlockSpec((_TN, _HP), lambda i: (i, 0)),
        out_shape=jax.ShapeDtypeStruct((_N, _HP), jnp.float32),
    )(h, agg, Wscl, Wlol)


def _final_body(h_ref, agg_ref, wsc_ref, wlo_ref, wout_ref, y_ref, d_ref,
                o_ref):
    f32 = jnp.float32
    h2 = (jnp.dot(h_ref[:, :_H], wsc_ref[...], preferred_element_type=f32)
          + jnp.dot(agg_ref[:, :_H], wlo_ref[...], preferred_element_type=f32))
    coeffs = jnp.dot(h2, wout_ref[...], preferred_element_type=f32)
    m = lax.dot_general(y_ref[...], d_ref[...], (((0,), (0,)), ((), ())),
                        preferred_element_type=f32)
    o_ref[...] = jnp.dot(coeffs, m, preferred_element_type=f32) * (
        1.0 / y_ref.shape[0])


def _final(h, agg, Wscl, Wlol, W_out, sphere_Y, sphere_dirs):
    s = sphere_Y.shape[0]
    return pl.pallas_call(
        _final_body,
        grid=(_N // _TN,),
        in_specs=[pl.BlockSpec((_TN, _HP), lambda i: (i, 0)),
                  pl.BlockSpec((_TN, _HP), lambda i: (i, 0)),
                  pl.BlockSpec((_H, _H), lambda i: (0, 0)),
                  pl.BlockSpec((_H, _H), lambda i: (0, 0)),
                  pl.BlockSpec((_H, _NSH), lambda i: (0, 0)),
                  pl.BlockSpec((s, _NSH), lambda i: (0, 0)),
                  pl.BlockSpec((s, 3), lambda i: (0, 0))],
        out_specs=pl.BlockSpec((_TN, 3), lambda i: (i, 0)),
        out_shape=jax.ShapeDtypeStruct((_N, 3), jnp.float32),
    )(h, agg, Wscl, Wlol, W_out, sphere_Y, sphere_dirs)


_G_CH = 40
_G_PER_W = _E // _NW


def _sc_gather(h, src):
    n_ch = _G_PER_W // _G_CH

    @functools.partial(
        pl.kernel,
        mesh=plsc.VectorSubcoreMesh(core_axis_name="c", subcore_axis_name="s"),
        out_type=jax.ShapeDtypeStruct((_E, _HP), jnp.float32),
        scratch_types=[
            pltpu.VMEM((_G_CH,), jnp.int32),
            pltpu.VMEM((_G_CH, _HP), jnp.float32),
            pltpu.SemaphoreType.DMA,
        ],
    )
    def k(h_hbm, idx_hbm, out_hbm, idx_v, rows_v, sem):
        wid = lax.axis_index("s") * _NC + lax.axis_index("c")
        base = wid * _G_PER_W

        def body(i, carry):
            off = base + i * _G_CH
            pltpu.sync_copy(idx_hbm.at[pl.ds(off, _G_CH)], idx_v)
            pltpu.async_copy(h_hbm.at[idx_v], rows_v, sem).wait()
            pltpu.sync_copy(rows_v, out_hbm.at[pl.ds(off, _G_CH)])
            return carry

        lax.fori_loop(0, n_ch, body, 0)

    return k(h, src)


_S_CH = 80
_S_PER_T = _E // _NS
_NP = 10240
_S_ROWS = _NP // _NS
_Z_ROWS = 32


def _sc_scatter(m, dst):
    n_ch = _S_PER_T // _S_CH

    @functools.partial(
        pl.kernel,
        mesh=plsc.VectorSubcoreMesh(core_axis_name="c", subcore_axis_name="s"),
        out_type=jax.ShapeDtypeStruct((_NP, _HP), jnp.float32),
        scratch_types=[
            pltpu.VMEM((_S_CH,), jnp.int32),
            pltpu.VMEM((_S_CH, 128), jnp.float32),
            pltpu.VMEM((_Z_ROWS, 128), jnp.float32),
            pltpu.VMEM_SHARED((_NP, 128), jnp.float32),
        ],
    )
    def k(m_hbm, dst_hbm, out_hbm, idx_v, rows_v, zbuf, acc):
        s = lax.axis_index("s")
        c = lax.axis_index("c")
        col = c * 128

        def zrow(r, carry):
            def zcol(kk, carry2):
                zbuf[r, pl.ds(kk * 16, 16)] = jnp.zeros((16,), jnp.float32)
                return carry2
            return lax.fori_loop(0, 128 // 16, zcol, carry)

        lax.fori_loop(0, _Z_ROWS, zrow, 0)

        def zcopy(j, carry):
            pltpu.sync_copy(zbuf,
                            acc.at[pl.ds(s * _S_ROWS + j * _Z_ROWS, _Z_ROWS)])
            return carry

        lax.fori_loop(0, _S_ROWS // _Z_ROWS, zcopy, 0)
        plsc.subcore_barrier()

        def body(i, carry):
            off = s * _S_PER_T + i * _S_CH
            pltpu.sync_copy(dst_hbm.at[pl.ds(off, _S_CH)], idx_v)
            pltpu.sync_copy(m_hbm.at[pl.ds(off, _S_CH), pl.ds(col, 128)],
                            rows_v)
            pltpu.sync_copy(rows_v, acc.at[idx_v], add=True)
            return carry

        lax.fori_loop(0, n_ch, body, 0)
        plsc.subcore_barrier()
        pltpu.sync_copy(acc.at[pl.ds(s * _S_ROWS, _S_ROWS)],
                        out_hbm.at[pl.ds(s * _S_ROWS, _S_ROWS),
                                   pl.ds(col, 128)])

    return k(m, dst)[:_N]


def kernel(x, edge_index, edge_attr, edge_sh, W_in, W_sc, W1, b1, W2, b2,
           W_tp, W_pg, W_lo, W_out, sphere_dirs, sphere_Y):
    src = edge_index[0]
    dst = edge_index[1]
    h = _node_in(x, W_in)
    for l in range(W_sc.shape[0]):
        xj = _sc_gather(h, src)
        m = _edge_msg(edge_attr, edge_sh, xj, W1[l], b1[l], W2[l], b2[l],
                      W_tp[l], W_pg[l])
        agg = _sc_scatter(m, dst)
        if l + 1 < W_sc.shape[0]:
            h = _node_update(h, agg, W_sc[l], W_lo[l])
        else:
            return _final(h, agg, W_sc[l], W_lo[l], W_out, sphere_Y,
                          sphere_dirs)


# speedup vs baseline: 2.4320x; 1.4486x over previous
import functools
import math

import jax
import jax.numpy as jnp
from jax import lax
from jax.experimental import pallas as pl
from jax.experimental.pallas import tpu as pltpu
from jax.experimental.pallas import tpu_sc as plsc

_N = 10000
_E = 160000
_H = 240
_HP = 256
_NSH = 9
_NB = 6
_CUT = 0.35
_TN = 2000
_TE = 2000
_C = 5
_CE = _E // _C

_NC = 2
_NS = 16
_NW = _NC * _NS


def _sigmoid(v):
    return 1.0 / (1.0 + jnp.exp(-v))


def _node_in_body(x_ref, w_ref, o_ref):
    h = jnp.dot(x_ref[...], w_ref[...], preferred_element_type=jnp.float32)
    o_ref[...] = jnp.concatenate(
        [h, jnp.zeros((h.shape[0], _HP - _H), jnp.float32)], axis=1)


def _node_in(x, W_in):
    d = x.shape[1]
    return pl.pallas_call(
        _node_in_body,
        grid=(_N // _TN,),
        in_specs=[pl.BlockSpec((_TN, d), lambda i: (i, 0)),
                  pl.BlockSpec((d, _H), lambda i: (0, 0))],
        out_specs=pl.BlockSpec((_TN, _HP), lambda i: (i, 0)),
        out_shape=jax.ShapeDtypeStruct((_N, _HP), jnp.float32),
    )(x, W_in)


def _edge_body(ea_ref, sh_ref, xj_ref, w1_ref, b1_ref, w2_ref, b2_ref,
               wtp_ref, wpg_ref, m_ref):
    f32 = jnp.float32
    ea = ea_ref[...]
    centers = lax.broadcasted_iota(jnp.int32, (1, _NB), 1).astype(f32) * (
        _CUT / (_NB - 1))
    width = _CUT / _NB * 0.5
    diff = (ea - centers) * (1.0 / width)
    rbf = jnp.exp(-0.5 * diff * diff)
    cut = 0.5 * (jnp.cos(ea * (math.pi / _CUT)) + 1.0)
    cut = cut * (ea < _CUT).astype(f32)
    rbf = rbf * cut
    t = jnp.dot(rbf, w1_ref[...], preferred_element_type=f32) + b1_ref[...]
    t = t * _sigmoid(t)
    tpw = jnp.dot(t, w2_ref[...], preferred_element_type=f32) + b2_ref[...]
    a = jnp.dot(sh_ref[...], wtp_ref[...], preferred_element_type=f32)
    u = xj_ref[:, :_H] * a * tpw
    pre = jnp.dot(u, wpg_ref[...], preferred_element_type=f32)
    scal = pre[:, :64]
    scal = scal * _sigmoid(scal)
    gates = _sigmoid(pre[:, 64:112])
    gated = pre[:, 112:288]
    ji = lax.broadcasted_iota(jnp.int32, (48, 176), 1)
    ii = lax.broadcasted_iota(jnp.int32, (48, 176), 0)
    gidx = jnp.where(ji < 96, ji // 3, 32 + (ji - 96) // 5)
    sel = (gidx == ii).astype(f32)
    g = jnp.dot(gates, sel, preferred_element_type=f32)
    pad = jnp.zeros((scal.shape[0], _HP - _H), f32)
    m_ref[...] = jnp.concatenate([scal, gated * g, pad], axis=1)


def _edge_msg(edge_attr, edge_sh, xj, W1l, b1l, W2l, b2l, Wtpl, Wpgl):
    ne = edge_attr.shape[0]
    return pl.pallas_call(
        _edge_body,
        grid=(ne // _TE,),
        in_specs=[
            pl.BlockSpec((_TE, 1), lambda i: (i, 0)),
            pl.BlockSpec((_TE, _NSH), lambda i: (i, 0)),
            pl.BlockSpec((_TE, _HP), lambda i: (i, 0)),
            pl.BlockSpec((_NB, 32), lambda i: (0, 0)),
            pl.BlockSpec((1, 32), lambda i: (0, 0)),
            pl.BlockSpec((32, _H), lambda i: (0, 0)),
            pl.BlockSpec((1, _H), lambda i: (0, 0)),
            pl.BlockSpec((_NSH, _H), lambda i: (0, 0)),
            pl.BlockSpec((_H, 288), lambda i: (0, 0)),
        ],
        out_specs=pl.BlockSpec((_TE, _HP), lambda i: (i, 0)),
        out_shape=jax.ShapeDtypeStruct((ne, _HP), jnp.float32),
        compiler_params=pltpu.CompilerParams(
            dimension_semantics=("arbitrary",)),
    )(edge_attr, edge_sh, xj, W1l, b1l.reshape(1, -1), W2l,
      b2l.reshape(1, -1), Wtpl, Wpgl)


def _node_update_body(*refs):
    f32 = jnp.float32
    h_ref = refs[0]
    agg_refs = refs[1:1 + _C]
    wsc_ref, wlo_ref, o_ref = refs[1 + _C:]
    agg = agg_refs[0][:, :_H]
    for r in agg_refs[1:]:
        agg = agg + r[:, :_H]
    h2 = (jnp.dot(h_ref[:, :_H], wsc_ref[...], preferred_element_type=f32)
          + jnp.dot(agg, wlo_ref[...], preferred_element_type=f32))
    o_ref[...] = jnp.concatenate(
        [h2, jnp.zeros((h2.shape[0], _HP - _H), f32)], axis=1)


def _node_update(h, aggs, Wscl, Wlol):
    return pl.pallas_call(
        _node_update_body,
        grid=(_N // _TN,),
        in_specs=[pl.BlockSpec((_TN, _HP), lambda i: (i, 0))]
        + [pl.BlockSpec((_TN, _HP), lambda i: (i, 0)) for _ in range(_C)]
        + [pl.BlockSpec((_H, _H), lambda i: (0, 0)),
           pl.BlockSpec((_H, _H), lambda i: (0, 0))],
        out_specs=pl.BlockSpec((_TN, _HP), lambda i: (i, 0)),
        out_shape=jax.ShapeDtypeStruct((_N, _HP), jnp.float32),
    )(h, *aggs, Wscl, Wlol)


def _final_body(*refs):
    f32 = jnp.float32
    h_ref = refs[0]
    agg_refs = refs[1:1 + _C]
    wsc_ref, wlo_ref, wout_ref, y_ref, d_ref, o_ref = refs[1 + _C:]
    agg = agg_refs[0][:, :_H]
    for r in agg_refs[1:]:
        agg = agg + r[:, :_H]
    h2 = (jnp.dot(h_ref[:, :_H], wsc_ref[...], preferred_element_type=f32)
          + jnp.dot(agg, wlo_ref[...], preferred_element_type=f32))
    coeffs = jnp.dot(h2, wout_ref[...], preferred_element_type=f32)
    m = lax.dot_general(y_ref[...], d_ref[...], (((0,), (0,)), ((), ())),
                        preferred_element_type=f32)
    o_ref[...] = jnp.dot(coeffs, m, preferred_element_type=f32) * (
        1.0 / y_ref.shape[0])


def _final(h, aggs, Wscl, Wlol, W_out, sphere_Y, sphere_dirs):
    s = sphere_Y.shape[0]
    return pl.pallas_call(
        _final_body,
        grid=(_N // _TN,),
        in_specs=[pl.BlockSpec((_TN, _HP), lambda i: (i, 0))]
        + [pl.BlockSpec((_TN, _HP), lambda i: (i, 0)) for _ in range(_C)]
        + [pl.BlockSpec((_H, _H), lambda i: (0, 0)),
           pl.BlockSpec((_H, _H), lambda i: (0, 0)),
           pl.BlockSpec((_H, _NSH), lambda i: (0, 0)),
           pl.BlockSpec((s, _NSH), lambda i: (0, 0)),
           pl.BlockSpec((s, 3), lambda i: (0, 0))],
        out_specs=pl.BlockSpec((_TN, 3), lambda i: (i, 0)),
        out_shape=jax.ShapeDtypeStruct((_N, 3), jnp.float32),
    )(h, *aggs, Wscl, Wlol, W_out, sphere_Y, sphere_dirs)


_G_CH = 40


def _sc_gather(h, src):
    ne = src.shape[0]
    per_w = ne // _NW
    n_ch = per_w // _G_CH

    @functools.partial(
        pl.kernel,
        mesh=plsc.VectorSubcoreMesh(core_axis_name="c", subcore_axis_name="s"),
        out_type=jax.ShapeDtypeStruct((ne, _HP), jnp.float32),
        scratch_types=[
            pltpu.VMEM((_G_CH,), jnp.int32),
            pltpu.VMEM((_G_CH, _HP), jnp.float32),
            pltpu.SemaphoreType.DMA,
        ],
    )
    def k(h_hbm, idx_hbm, out_hbm, idx_v, rows_v, sem):
        wid = lax.axis_index("s") * _NC + lax.axis_index("c")
        base = wid * per_w

        def body(i, carry):
            off = base + i * _G_CH
            pltpu.sync_copy(idx_hbm.at[pl.ds(off, _G_CH)], idx_v)
            pltpu.async_copy(h_hbm.at[idx_v], rows_v, sem).wait()
            pltpu.sync_copy(rows_v, out_hbm.at[pl.ds(off, _G_CH)])
            return carry

        lax.fori_loop(0, n_ch, body, 0)

    return k(h, src)


_S_CH = 80
_NP = 10240
_S_ROWS = _NP // _NS
_Z_ROWS = 32


def _sc_scatter(m, dst):
    ne = dst.shape[0]
    per_t = ne // _NS
    n_ch = per_t // _S_CH

    @functools.partial(
        pl.kernel,
        mesh=plsc.VectorSubcoreMesh(core_axis_name="c", subcore_axis_name="s"),
        out_type=jax.ShapeDtypeStruct((_NP, _HP), jnp.float32),
        scratch_types=[
            pltpu.VMEM((_S_CH,), jnp.int32),
            pltpu.VMEM((_S_CH, 128), jnp.float32),
            pltpu.VMEM((_Z_ROWS, 128), jnp.float32),
            pltpu.VMEM_SHARED((_NP, 128), jnp.float32),
        ],
    )
    def k(m_hbm, dst_hbm, out_hbm, idx_v, rows_v, zbuf, acc):
        s = lax.axis_index("s")
        c = lax.axis_index("c")
        col = c * 128

        def zrow(r, carry):
            def zcol(kk, carry2):
                zbuf[r, pl.ds(kk * 16, 16)] = jnp.zeros((16,), jnp.float32)
                return carry2
            return lax.fori_loop(0, 128 // 16, zcol, carry)

        lax.fori_loop(0, _Z_ROWS, zrow, 0)

        def zcopy(j, carry):
            pltpu.sync_copy(zbuf,
                            acc.at[pl.ds(s * _S_ROWS + j * _Z_ROWS, _Z_ROWS)])
            return carry

        lax.fori_loop(0, _S_ROWS // _Z_ROWS, zcopy, 0)
        plsc.subcore_barrier()

        def body(i, carry):
            off = s * per_t + i * _S_CH
            pltpu.sync_copy(dst_hbm.at[pl.ds(off, _S_CH)], idx_v)
            pltpu.sync_copy(m_hbm.at[pl.ds(off, _S_CH), pl.ds(col, 128)],
                            rows_v)
            pltpu.sync_copy(rows_v, acc.at[idx_v], add=True)
            return carry

        lax.fori_loop(0, n_ch, body, 0)
        plsc.subcore_barrier()
        pltpu.sync_copy(acc.at[pl.ds(s * _S_ROWS, _S_ROWS)],
                        out_hbm.at[pl.ds(s * _S_ROWS, _S_ROWS),
                                   pl.ds(col, 128)])

    return k(m, dst)


def kernel(x, edge_index, edge_attr, edge_sh, W_in, W_sc, W1, b1, W2, b2,
           W_tp, W_pg, W_lo, W_out, sphere_dirs, sphere_Y):
    src = edge_index[0]
    dst = edge_index[1]
    srcs = [src[i * _CE:(i + 1) * _CE] for i in range(_C)]
    dsts = [dst[i * _CE:(i + 1) * _CE] for i in range(_C)]
    eas = [edge_attr[i * _CE:(i + 1) * _CE] for i in range(_C)]
    shs = [edge_sh[i * _CE:(i + 1) * _CE] for i in range(_C)]
    h = _node_in(x, W_in)
    for l in range(W_sc.shape[0]):
        aggs = []
        for i in range(_C):
            xj = _sc_gather(h, srcs[i])
            m = _edge_msg(eas[i], shs[i], xj, W1[l], b1[l], W2[l], b2[l],
                          W_tp[l], W_pg[l])
            aggs.append(_sc_scatter(m, dsts[i]))
        if l + 1 < W_sc.shape[0]:
            h = _node_update(h, aggs, W_sc[l], W_lo[l])
        else:
            return _final(h, aggs, W_sc[l], W_lo[l], W_out, sphere_Y,
                          sphere_dirs)


# speedup vs baseline: 2.4344x; 1.0010x over previous
import functools
import math

import jax
import jax.numpy as jnp
from jax import lax
from jax.experimental import pallas as pl
from jax.experimental.pallas import tpu as pltpu
from jax.experimental.pallas import tpu_sc as plsc

_N = 10000
_E = 160000
_H = 240
_HP = 256
_NSH = 9
_NB = 6
_CUT = 0.35
_TN = 2000
_TE = 2000
_C = 5
_CE = _E // _C

_NC = 2
_NS = 16
_NW = _NC * _NS


def _sigmoid(v):
    return 1.0 / (1.0 + jnp.exp(-v))


def _node_in_body(x_ref, w_ref, o_ref):
    h = jnp.dot(x_ref[...], w_ref[...], preferred_element_type=jnp.float32)
    o_ref[...] = jnp.concatenate(
        [h, jnp.zeros((h.shape[0], _HP - _H), jnp.float32)], axis=1)


def _node_in(x, W_in):
    d = x.shape[1]
    return pl.pallas_call(
        _node_in_body,
        grid=(_N // _TN,),
        in_specs=[pl.BlockSpec((_TN, d), lambda i: (i, 0)),
                  pl.BlockSpec((d, _H), lambda i: (0, 0))],
        out_specs=pl.BlockSpec((_TN, _HP), lambda i: (i, 0)),
        out_shape=jax.ShapeDtypeStruct((_N, _HP), jnp.float32),
    )(x, W_in)


def _edge_body(ea_ref, sh_ref, xj_ref, w1_ref, b1_ref, w2_ref, b2_ref,
               wtp_ref, wpg_ref, m_ref):
    f32 = jnp.float32
    ea = ea_ref[...]
    centers = lax.broadcasted_iota(jnp.int32, (1, _NB), 1).astype(f32) * (
        _CUT / (_NB - 1))
    width = _CUT / _NB * 0.5
    diff = (ea - centers) * (1.0 / width)
    rbf = jnp.exp(-0.5 * diff * diff)
    cut = 0.5 * (jnp.cos(ea * (math.pi / _CUT)) + 1.0)
    cut = cut * (ea < _CUT).astype(f32)
    rbf = rbf * cut
    t = jnp.dot(rbf, w1_ref[...], preferred_element_type=f32) + b1_ref[...]
    t = t * _sigmoid(t)
    tpw = jnp.dot(t, w2_ref[...], preferred_element_type=f32) + b2_ref[...]
    a = jnp.dot(sh_ref[...], wtp_ref[...], preferred_element_type=f32)
    xj = xj_ref[:, :_H].astype(f32)
    u = xj * a * tpw
    pre = jnp.dot(u.astype(jnp.bfloat16), wpg_ref[...],
                  preferred_element_type=f32)
    scal = pre[:, :64]
    scal = scal * _sigmoid(scal)
    gates = _sigmoid(pre[:, 64:112])
    gated = pre[:, 112:288]
    ji = lax.broadcasted_iota(jnp.int32, (48, 176), 1)
    ii = lax.broadcasted_iota(jnp.int32, (48, 176), 0)
    gidx = jnp.where(ji < 96, ji // 3, 32 + (ji - 96) // 5)
    sel = (gidx == ii).astype(f32)
    g = jnp.dot(gates, sel, preferred_element_type=f32)
    pad = jnp.zeros((scal.shape[0], _HP - _H), f32)
    m_ref[...] = jnp.concatenate([scal, gated * g, pad], axis=1)


def _edge_msg(edge_attr, edge_sh, xj, W1l, b1l, W2l, b2l, Wtpl, Wpgl):
    ne = edge_attr.shape[0]
    return pl.pallas_call(
        _edge_body,
        grid=(ne // _TE,),
        in_specs=[
            pl.BlockSpec((_TE, 1), lambda i: (i, 0)),
            pl.BlockSpec((_TE, _NSH), lambda i: (i, 0)),
            pl.BlockSpec((_TE, _HP), lambda i: (i, 0)),
            pl.BlockSpec((_NB, 32), lambda i: (0, 0)),
            pl.BlockSpec((1, 32), lambda i: (0, 0)),
            pl.BlockSpec((32, _H), lambda i: (0, 0)),
            pl.BlockSpec((1, _H), lambda i: (0, 0)),
            pl.BlockSpec((_NSH, _H), lambda i: (0, 0)),
            pl.BlockSpec((_H, 288), lambda i: (0, 0)),
        ],
        out_specs=pl.BlockSpec((_TE, _HP), lambda i: (i, 0)),
        out_shape=jax.ShapeDtypeStruct((ne, _HP), jnp.float32),
        compiler_params=pltpu.CompilerParams(
            dimension_semantics=("arbitrary",)),
    )(edge_attr, edge_sh, xj, W1l, b1l.reshape(1, -1), W2l,
      b2l.reshape(1, -1), Wtpl, Wpgl.astype(jnp.bfloat16))


def _node_update_body(*refs):
    f32 = jnp.float32
    h_ref = refs[0]
    agg_refs = refs[1:1 + _C]
    wsc_ref, wlo_ref, o_ref = refs[1 + _C:]
    agg = agg_refs[0][:, :_H]
    for r in agg_refs[1:]:
        agg = agg + r[:, :_H]
    h2 = (jnp.dot(h_ref[:, :_H], wsc_ref[...], preferred_element_type=f32)
          + jnp.dot(agg, wlo_ref[...], preferred_element_type=f32))
    o_ref[...] = jnp.concatenate(
        [h2, jnp.zeros((h2.shape[0], _HP - _H), f32)], axis=1)


def _node_update(h, aggs, Wscl, Wlol):
    return pl.pallas_call(
        _node_update_body,
        grid=(_N // _TN,),
        in_specs=[pl.BlockSpec((_TN, _HP), lambda i: (i, 0))]
        + [pl.BlockSpec((_TN, _HP), lambda i: (i, 0)) for _ in range(_C)]
        + [pl.BlockSpec((_H, _H), lambda i: (0, 0)),
           pl.BlockSpec((_H, _H), lambda i: (0, 0))],
        out_specs=pl.BlockSpec((_TN, _HP), lambda i: (i, 0)),
        out_shape=jax.ShapeDtypeStruct((_N, _HP), jnp.float32),
    )(h, *aggs, Wscl, Wlol)


def _final_body(*refs):
    f32 = jnp.float32
    h_ref = refs[0]
    agg_refs = refs[1:1 + _C]
    wsc_ref, wlo_ref, wout_ref, y_ref, d_ref, o_ref = refs[1 + _C:]
    agg = agg_refs[0][:, :_H]
    for r in agg_refs[1:]:
        agg = agg + r[:, :_H]
    h2 = (jnp.dot(h_ref[:, :_H], wsc_ref[...], preferred_element_type=f32)
          + jnp.dot(agg, wlo_ref[...], preferred_element_type=f32))
    coeffs = jnp.dot(h2, wout_ref[...], preferred_element_type=f32)
    m = lax.dot_general(y_ref[...], d_ref[...], (((0,), (0,)), ((), ())),
                        preferred_element_type=f32)
    o_ref[...] = jnp.dot(coeffs, m, preferred_element_type=f32) * (
        1.0 / y_ref.shape[0])


def _final(h, aggs, Wscl, Wlol, W_out, sphere_Y, sphere_dirs):
    s = sphere_Y.shape[0]
    return pl.pallas_call(
        _final_body,
        grid=(_N // _TN,),
        in_specs=[pl.BlockSpec((_TN, _HP), lambda i: (i, 0))]
        + [pl.BlockSpec((_TN, _HP), lambda i: (i, 0)) for _ in range(_C)]
        + [pl.BlockSpec((_H, _H), lambda i: (0, 0)),
           pl.BlockSpec((_H, _H), lambda i: (0, 0)),
           pl.BlockSpec((_H, _NSH), lambda i: (0, 0)),
           pl.BlockSpec((s, _NSH), lambda i: (0, 0)),
           pl.BlockSpec((s, 3), lambda i: (0, 0))],
        out_specs=pl.BlockSpec((_TN, 3), lambda i: (i, 0)),
        out_shape=jax.ShapeDtypeStruct((_N, 3), jnp.float32),
    )(h, *aggs, Wscl, Wlol, W_out, sphere_Y, sphere_dirs)


_G_CH = 40


def _sc_gather(h, src):
    ne = src.shape[0]
    per_w = ne // _NW
    n_ch = per_w // _G_CH

    @functools.partial(
        pl.kernel,
        mesh=plsc.VectorSubcoreMesh(core_axis_name="c", subcore_axis_name="s"),
        out_type=jax.ShapeDtypeStruct((ne, _HP), jnp.float32),
        scratch_types=[
            pltpu.VMEM((_G_CH,), jnp.int32),
            pltpu.VMEM((_G_CH, _HP), jnp.float32),
            pltpu.SemaphoreType.DMA,
        ],
    )
    def k(h_hbm, idx_hbm, out_hbm, idx_v, rows_v, sem):
        wid = lax.axis_index("s") * _NC + lax.axis_index("c")
        base = wid * per_w

        def body(i, carry):
            off = base + i * _G_CH
            pltpu.sync_copy(idx_hbm.at[pl.ds(off, _G_CH)], idx_v)
            pltpu.async_copy(h_hbm.at[idx_v], rows_v, sem).wait()
            pltpu.sync_copy(rows_v, out_hbm.at[pl.ds(off, _G_CH)])
            return carry

        lax.fori_loop(0, n_ch, body, 0)

    return k(h, src)


_S_CH = 80
_NP = 10240
_S_ROWS = _NP // _NS
_Z_ROWS = 32


def _sc_scatter(m, dst):
    ne = dst.shape[0]
    per_t = ne // _NS
    n_ch = per_t // _S_CH

    @functools.partial(
        pl.kernel,
        mesh=plsc.VectorSubcoreMesh(core_axis_name="c", subcore_axis_name="s"),
        out_type=jax.ShapeDtypeStruct((_NP, _HP), jnp.float32),
        scratch_types=[
            pltpu.VMEM((_S_CH,), jnp.int32),
            pltpu.VMEM((_S_CH, 128), jnp.float32),
            pltpu.VMEM((_Z_ROWS, 128), jnp.float32),
            pltpu.VMEM_SHARED((_NP, 128), jnp.float32),
        ],
    )
    def k(m_hbm, dst_hbm, out_hbm, idx_v, rows_v, zbuf, acc):
        s = lax.axis_index("s")
        c = lax.axis_index("c")
        col = c * 128

        def zrow(r, carry):
            def zcol(kk, carry2):
                zbuf[r, pl.ds(kk * 16, 16)] = jnp.zeros((16,), jnp.float32)
                return carry2
            return lax.fori_loop(0, 128 // 16, zcol, carry)

        lax.fori_loop(0, _Z_ROWS, zrow, 0)

        def zcopy(j, carry):
            pltpu.sync_copy(zbuf,
                            acc.at[pl.ds(s * _S_ROWS + j * _Z_ROWS, _Z_ROWS)])
            return carry

        lax.fori_loop(0, _S_ROWS // _Z_ROWS, zcopy, 0)
        plsc.subcore_barrier()

        def body(i, carry):
            off = s * per_t + i * _S_CH
            pltpu.sync_copy(dst_hbm.at[pl.ds(off, _S_CH)], idx_v)
            pltpu.sync_copy(m_hbm.at[pl.ds(off, _S_CH), pl.ds(col, 128)],
                            rows_v)
            pltpu.sync_copy(rows_v, acc.at[idx_v], add=True)
            return carry

        lax.fori_loop(0, n_ch, body, 0)
        plsc.subcore_barrier()
        pltpu.sync_copy(acc.at[pl.ds(s * _S_ROWS, _S_ROWS)],
                        out_hbm.at[pl.ds(s * _S_ROWS, _S_ROWS),
                                   pl.ds(col, 128)])

    return k(m, dst)


def kernel(x, edge_index, edge_attr, edge_sh, W_in, W_sc, W1, b1, W2, b2,
           W_tp, W_pg, W_lo, W_out, sphere_dirs, sphere_Y):
    src = edge_index[0]
    dst = edge_index[1]
    srcs = [src[i * _CE:(i + 1) * _CE] for i in range(_C)]
    dsts = [dst[i * _CE:(i + 1) * _CE] for i in range(_C)]
    eas = [edge_attr[i * _CE:(i + 1) * _CE] for i in range(_C)]
    shs = [edge_sh[i * _CE:(i + 1) * _CE] for i in range(_C)]
    h = _node_in(x, W_in)
    for l in range(W_sc.shape[0]):
        aggs = []
        for i in range(_C):
            xj = _sc_gather(h, srcs[i])
            m = _edge_msg(eas[i], shs[i], xj, W1[l], b1[l], W2[l], b2[l],
                          W_tp[l], W_pg[l])
            aggs.append(_sc_scatter(m, dsts[i]))
        if l + 1 < W_sc.shape[0]:
            h = _node_update(h, aggs, W_sc[l], W_lo[l])
        else:
            return _final(h, aggs, W_sc[l], W_lo[l], W_out, sphere_Y,
                          sphere_dirs)


# speedup vs baseline: 2.5232x; 1.0365x over previous
import functools
import math

import jax
import jax.numpy as jnp
from jax import lax
from jax.experimental import pallas as pl
from jax.experimental.pallas import tpu as pltpu
from jax.experimental.pallas import tpu_sc as plsc

_N = 10000
_E = 160000
_H = 240
_HP = 256
_NSH = 9
_NB = 6
_CUT = 0.35
_TN = 2000
_TE = 2000
_C = 5
_CE = _E // _C

_NC = 2
_NS = 16
_NW = _NC * _NS


def _sigmoid(v):
    return 1.0 / (1.0 + jnp.exp(-v))


def _pack(hp):
    bits = lax.bitcast_convert_type(hp, jnp.uint32)
    lo = lax.shift_right_logical(bits[:, :128], jnp.uint32(16))
    hi = jnp.bitwise_and(bits[:, 128:], jnp.uint32(0xFFFF0000))
    return lax.bitcast_convert_type(jnp.bitwise_or(lo, hi), jnp.int32)


def _unpack(packed):
    bits = lax.bitcast_convert_type(packed, jnp.uint32)
    lo = lax.bitcast_convert_type(
        lax.shift_left(bits, jnp.uint32(16)), jnp.float32)
    hi = lax.bitcast_convert_type(
        jnp.bitwise_and(bits, jnp.uint32(0xFFFF0000)), jnp.float32)
    return jnp.concatenate([lo, hi[:, :_H - 128]], axis=1)


def _node_in_body(x_ref, w_ref, o_ref, op_ref):
    h = jnp.dot(x_ref[...], w_ref[...], preferred_element_type=jnp.float32)
    hp = jnp.concatenate(
        [h, jnp.zeros((h.shape[0], _HP - _H), jnp.float32)], axis=1)
    o_ref[...] = hp
    op_ref[...] = _pack(hp)


def _node_in(x, W_in):
    d = x.shape[1]
    return pl.pallas_call(
        _node_in_body,
        grid=(_N // _TN,),
        in_specs=[pl.BlockSpec((_TN, d), lambda i: (i, 0)),
                  pl.BlockSpec((d, _H), lambda i: (0, 0))],
        out_specs=[pl.BlockSpec((_TN, _HP), lambda i: (i, 0)),
                   pl.BlockSpec((_TN, 128), lambda i: (i, 0))],
        out_shape=[jax.ShapeDtypeStruct((_N, _HP), jnp.float32),
                   jax.ShapeDtypeStruct((_N, 128), jnp.int32)],
    )(x, W_in)


def _msg240(ea, sh_ref, xj, w1_ref, b1_ref, w2_ref, b2_ref, wtp_ref, wpg_ref):
    f32 = jnp.float32
    centers = lax.broadcasted_iota(jnp.int32, (1, _NB), 1).astype(f32) * (
        _CUT / (_NB - 1))
    width = _CUT / _NB * 0.5
    diff = (ea - centers) * (1.0 / width)
    rbf = jnp.exp(-0.5 * diff * diff)
    cut = 0.5 * (jnp.cos(ea * (math.pi / _CUT)) + 1.0)
    cut = cut * (ea < _CUT).astype(f32)
    rbf = rbf * cut
    t = jnp.dot(rbf, w1_ref[...], preferred_element_type=f32) + b1_ref[...]
    t = t * _sigmoid(t)
    tpw = jnp.dot(t, w2_ref[...], preferred_element_type=f32) + b2_ref[...]
    a = jnp.dot(sh_ref[...], wtp_ref[...], preferred_element_type=f32)
    u = xj * a * tpw
    pre = jnp.dot(u.astype(jnp.bfloat16), wpg_ref[...],
                  preferred_element_type=f32)
    scal = pre[:, :64]
    scal = scal * _sigmoid(scal)
    gates = _sigmoid(pre[:, 64:112])
    gated = pre[:, 112:288]
    ji = lax.broadcasted_iota(jnp.int32, (48, 176), 1)
    ii = lax.broadcasted_iota(jnp.int32, (48, 176), 0)
    gidx = jnp.where(ji < 96, ji // 3, 32 + (ji - 96) // 5)
    sel = (gidx == ii).astype(f32)
    g = jnp.dot(gates, sel, preferred_element_type=f32)
    return jnp.concatenate([scal, gated * g], axis=1)


def _edge_body_w(ea_ref, sh_ref, xj_ref, w1_ref, b1_ref, w2_ref, b2_ref,
                 wtp_ref, wpg_ref, m_ref):
    xj = _unpack(xj_ref[...])
    msg = _msg240(ea_ref[...], sh_ref, xj, w1_ref, b1_ref, w2_ref, b2_ref,
                  wtp_ref, wpg_ref)
    pad = jnp.zeros((msg.shape[0], _HP - _H), jnp.float32)
    m_ref[...] = jnp.concatenate([msg, pad], axis=1)


def _edge_msg_w(edge_attr, edge_sh, xj, W1l, b1l, W2l, b2l, Wtpl, Wpgl):
    ne = edge_attr.shape[0]
    return pl.pallas_call(
        _edge_body_w,
        grid=(ne // _TE,),
        in_specs=[
            pl.BlockSpec((_TE, 1), lambda i: (i, 0)),
            pl.BlockSpec((_TE, _NSH), lambda i: (i, 0)),
            pl.BlockSpec((_TE, 128), lambda i: (i, 0)),
            pl.BlockSpec((_NB, 32), lambda i: (0, 0)),
            pl.BlockSpec((1, 32), lambda i: (0, 0)),
            pl.BlockSpec((32, _H), lambda i: (0, 0)),
            pl.BlockSpec((1, _H), lambda i: (0, 0)),
            pl.BlockSpec((_NSH, _H), lambda i: (0, 0)),
            pl.BlockSpec((_H, 288), lambda i: (0, 0)),
        ],
        out_specs=pl.BlockSpec((_TE, _HP), lambda i: (i, 0)),
        out_shape=jax.ShapeDtypeStruct((ne, _HP), jnp.float32),
        compiler_params=pltpu.CompilerParams(
            dimension_semantics=("arbitrary",)),
    )(edge_attr, edge_sh, xj, W1l, b1l.reshape(1, -1), W2l,
      b2l.reshape(1, -1), Wtpl, Wpgl.astype(jnp.bfloat16))


def _edge_body_l2(ea_ref, sh_ref, xj_ref, w1_ref, b1_ref, w2_ref, b2_ref,
                  wtp_ref, wpg_ref, wlo_ref, wout_ref, m_ref):
    f32 = jnp.float32
    xj = _unpack(xj_ref[...])
    msg = _msg240(ea_ref[...], sh_ref, xj, w1_ref, b1_ref, w2_ref, b2_ref,
                  wtp_ref, wpg_ref)
    wlw = jnp.dot(wlo_ref[...], wout_ref[...], preferred_element_type=f32)
    m9 = jnp.dot(msg, wlw, preferred_element_type=f32)
    pad = jnp.zeros((m9.shape[0], 128 - _NSH), f32)
    m_ref[...] = jnp.concatenate([m9, pad], axis=1)


def _edge_msg_l2(edge_attr, edge_sh, xj, W1l, b1l, W2l, b2l, Wtpl, Wpgl,
                 Wlol, W_out):
    ne = edge_attr.shape[0]
    return pl.pallas_call(
        _edge_body_l2,
        grid=(ne // _TE,),
        in_specs=[
            pl.BlockSpec((_TE, 1), lambda i: (i, 0)),
            pl.BlockSpec((_TE, _NSH), lambda i: (i, 0)),
            pl.BlockSpec((_TE, 128), lambda i: (i, 0)),
            pl.BlockSpec((_NB, 32), lambda i: (0, 0)),
            pl.BlockSpec((1, 32), lambda i: (0, 0)),
            pl.BlockSpec((32, _H), lambda i: (0, 0)),
            pl.BlockSpec((1, _H), lambda i: (0, 0)),
            pl.BlockSpec((_NSH, _H), lambda i: (0, 0)),
            pl.BlockSpec((_H, 288), lambda i: (0, 0)),
            pl.BlockSpec((_H, _H), lambda i: (0, 0)),
            pl.BlockSpec((_H, _NSH), lambda i: (0, 0)),
        ],
        out_specs=pl.BlockSpec((_TE, 128), lambda i: (i, 0)),
        out_shape=jax.ShapeDtypeStruct((ne, 128), jnp.float32),
        compiler_params=pltpu.CompilerParams(
            dimension_semantics=("arbitrary",)),
    )(edge_attr, edge_sh, xj, W1l, b1l.reshape(1, -1), W2l,
      b2l.reshape(1, -1), Wtpl, Wpgl.astype(jnp.bfloat16), Wlol, W_out)


def _node_update_body(*refs):
    f32 = jnp.float32
    h_ref = refs[0]
    agg_refs = refs[1:1 + _C]
    wsc_ref, wlo_ref, o_ref, op_ref = refs[1 + _C:]
    agg = agg_refs[0][:, :_H]
    for r in agg_refs[1:]:
        agg = agg + r[:, :_H]
    h2 = (jnp.dot(h_ref[:, :_H], wsc_ref[...], preferred_element_type=f32)
          + jnp.dot(agg, wlo_ref[...], preferred_element_type=f32))
    hp = jnp.concatenate(
        [h2, jnp.zeros((h2.shape[0], _HP - _H), f32)], axis=1)
    o_ref[...] = hp
    op_ref[...] = _pack(hp)


def _node_update(h, aggs, Wscl, Wlol):
    return pl.pallas_call(
        _node_update_body,
        grid=(_N // _TN,),
        in_specs=[pl.BlockSpec((_TN, _HP), lambda i: (i, 0))]
        + [pl.BlockSpec((_TN, _HP), lambda i: (i, 0)) for _ in range(_C)]
        + [pl.BlockSpec((_H, _H), lambda i: (0, 0)),
           pl.BlockSpec((_H, _H), lambda i: (0, 0))],
        out_specs=[pl.BlockSpec((_TN, _HP), lambda i: (i, 0)),
                   pl.BlockSpec((_TN, 128), lambda i: (i, 0))],
        out_shape=[jax.ShapeDtypeStruct((_N, _HP), jnp.float32),
                   jax.ShapeDtypeStruct((_N, 128), jnp.int32)],
    )(h, *aggs, Wscl, Wlol)


def _final_body(*refs):
    f32 = jnp.float32
    h_ref = refs[0]
    agg_refs = refs[1:1 + 2 * _C]
    wsc_ref, wout_ref, y_ref, d_ref, o_ref = refs[1 + 2 * _C:]
    agg = agg_refs[0][...]
    for r in agg_refs[1:]:
        agg = agg + r[...]
    h2 = jnp.dot(h_ref[:, :_H], wsc_ref[...], preferred_element_type=f32)
    coeffs = jnp.dot(h2, wout_ref[...],
                     preferred_element_type=f32) + agg[:, :_NSH]
    m = lax.dot_general(y_ref[...], d_ref[...], (((0,), (0,)), ((), ())),
                        preferred_element_type=f32)
    o_ref[...] = jnp.dot(coeffs, m, preferred_element_type=f32) * (
        1.0 / y_ref.shape[0])


def _final(h, aggs, Wscl, W_out, sphere_Y, sphere_dirs):
    s = sphere_Y.shape[0]
    return pl.pallas_call(
        _final_body,
        grid=(_N // _TN,),
        in_specs=[pl.BlockSpec((_TN, _HP), lambda i: (i, 0))]
        + [pl.BlockSpec((_TN, 128), lambda i: (i, 0)) for _ in aggs]
        + [pl.BlockSpec((_H, _H), lambda i: (0, 0)),
           pl.BlockSpec((_H, _NSH), lambda i: (0, 0)),
           pl.BlockSpec((s, _NSH), lambda i: (0, 0)),
           pl.BlockSpec((s, 3), lambda i: (0, 0))],
        out_specs=pl.BlockSpec((_TN, 3), lambda i: (i, 0)),
        out_shape=jax.ShapeDtypeStruct((_N, 3), jnp.float32),
    )(h, *aggs, Wscl, W_out, sphere_Y, sphere_dirs)


_G_CHW = 40
_G_CHX = 200


def _sc_gather(h, src, chunk):
    ne = src.shape[0]
    width = h.shape[1]
    per_w = ne // _NW
    n_ch = per_w // chunk

    @functools.partial(
        pl.kernel,
        mesh=plsc.VectorSubcoreMesh(core_axis_name="c", subcore_axis_name="s"),
        out_type=jax.ShapeDtypeStruct((ne, width), h.dtype),
        scratch_types=[
            pltpu.VMEM((chunk,), jnp.int32),
            pltpu.VMEM((chunk, width), h.dtype),
            pltpu.SemaphoreType.DMA,
        ],
    )
    def k(h_hbm, idx_hbm, out_hbm, idx_v, rows_v, sem):
        wid = lax.axis_index("s") * _NC + lax.axis_index("c")
        base = wid * per_w

        def body(i, carry):
            off = base + i * chunk
            pltpu.sync_copy(idx_hbm.at[pl.ds(off, chunk)], idx_v)
            pltpu.async_copy(h_hbm.at[idx_v], rows_v, sem).wait()
            pltpu.sync_copy(rows_v, out_hbm.at[pl.ds(off, chunk)])
            return carry

        lax.fori_loop(0, n_ch, body, 0)

    return k(h, src)


_S_CH = 80
_NP = 10240
_S_ROWS = _NP // _NS
_Z_ROWS = 32


def _sc_scatter(m, dst):
    ne = dst.shape[0]
    per_t = ne // _NS
    n_ch = per_t // _S_CH

    @functools.partial(
        pl.kernel,
        mesh=plsc.VectorSubcoreMesh(core_axis_name="c", subcore_axis_name="s"),
        out_type=jax.ShapeDtypeStruct((_NP, _HP), jnp.float32),
        scratch_types=[
            pltpu.VMEM((_S_CH,), jnp.int32),
            pltpu.VMEM((_S_CH, 128), jnp.float32),
            pltpu.VMEM((_Z_ROWS, 128), jnp.float32),
            pltpu.VMEM_SHARED((_NP, 128), jnp.float32),
        ],
    )
    def k(m_hbm, dst_hbm, out_hbm, idx_v, rows_v, zbuf, acc):
        s = lax.axis_index("s")
        c = lax.axis_index("c")
        col = c * 128

        def zrow(r, carry):
            def zcol(kk, carry2):
                zbuf[r, pl.ds(kk * 16, 16)] = jnp.zeros((16,), jnp.float32)
                return carry2
            return lax.fori_loop(0, 128 // 16, zcol, carry)

        lax.fori_loop(0, _Z_ROWS, zrow, 0)

        def zcopy(j, carry):
            pltpu.sync_copy(zbuf,
                            acc.at[pl.ds(s * _S_ROWS + j * _Z_ROWS, _Z_ROWS)])
            return carry

        lax.fori_loop(0, _S_ROWS // _Z_ROWS, zcopy, 0)
        plsc.subcore_barrier()

        def body(i, carry):
            off = s * per_t + i * _S_CH
            pltpu.sync_copy(dst_hbm.at[pl.ds(off, _S_CH)], idx_v)
            pltpu.sync_copy(m_hbm.at[pl.ds(off, _S_CH), pl.ds(col, 128)],
                            rows_v)
            pltpu.sync_copy(rows_v, acc.at[idx_v], add=True)
            return carry

        lax.fori_loop(0, n_ch, body, 0)
        plsc.subcore_barrier()
        pltpu.sync_copy(acc.at[pl.ds(s * _S_ROWS, _S_ROWS)],
                        out_hbm.at[pl.ds(s * _S_ROWS, _S_ROWS),
                                   pl.ds(col, 128)])

    return k(m, dst)


_S9_CH = 40


def _sc_scatter9(m, dst):
    ne = dst.shape[0]
    half = ne // _NC
    per_t = half // _NS
    n_ch = per_t // _S9_CH

    @functools.partial(
        pl.kernel,
        mesh=plsc.VectorSubcoreMesh(core_axis_name="c", subcore_axis_name="s"),
        out_type=jax.ShapeDtypeStruct((2 * _NP, 128), jnp.float32),
        scratch_types=[
            pltpu.VMEM((_S9_CH,), jnp.int32),
            pltpu.VMEM((_S9_CH, 128), jnp.float32),
            pltpu.VMEM((_Z_ROWS, 128), jnp.float32),
            pltpu.VMEM_SHARED((_NP, 128), jnp.float32),
        ],
    )
    def k(m_hbm, dst_hbm, out_hbm, idx_v, rows_v, zbuf, acc):
        s = lax.axis_index("s")
        c = lax.axis_index("c")

        def zrow(r, carry):
            def zcol(kk, carry2):
                zbuf[r, pl.ds(kk * 16, 16)] = jnp.zeros((16,), jnp.float32)
                return carry2
            return lax.fori_loop(0, 128 // 16, zcol, carry)

        lax.fori_loop(0, _Z_ROWS, zrow, 0)

        def zcopy(j, carry):
            pltpu.sync_copy(
                zbuf, acc.at[pl.ds(s * _S_ROWS + j * _Z_ROWS, _Z_ROWS)])
            return carry

        lax.fori_loop(0, _S_ROWS // _Z_ROWS, zcopy, 0)
        plsc.subcore_barrier()

        def body(i, carry):
            off = c * half + s * per_t + i * _S9_CH
            pltpu.sync_copy(dst_hbm.at[pl.ds(off, _S9_CH)], idx_v)
            pltpu.sync_copy(m_hbm.at[pl.ds(off, _S9_CH)], rows_v)
            pltpu.sync_copy(rows_v, acc.at[idx_v], add=True)
            return carry

        lax.fori_loop(0, n_ch, body, 0)
        plsc.subcore_barrier()
        pltpu.sync_copy(acc.at[pl.ds(s * _S_ROWS, _S_ROWS)],
                        out_hbm.at[pl.ds(c * _NP + s * _S_ROWS, _S_ROWS)])

    return k(m, dst)


def kernel(x, edge_index, edge_attr, edge_sh, W_in, W_sc, W1, b1, W2, b2,
           W_tp, W_pg, W_lo, W_out, sphere_dirs, sphere_Y):
    src = edge_index[0]
    dst = edge_index[1]
    srcs = [src[i * _CE:(i + 1) * _CE] for i in range(_C)]
    dsts = [dst[i * _CE:(i + 1) * _CE] for i in range(_C)]
    eas = [edge_attr[i * _CE:(i + 1) * _CE] for i in range(_C)]
    shs = [edge_sh[i * _CE:(i + 1) * _CE] for i in range(_C)]
    h, hpk = _node_in(x, W_in)
    aggs = []
    for i in range(_C):
        xj = _sc_gather(hpk, srcs[i], _G_CHW)
        m = _edge_msg_w(eas[i], shs[i], xj, W1[0], b1[0], W2[0],
                        b2[0], W_tp[0], W_pg[0])
        aggs.append(_sc_scatter(m, dsts[i]))
    h, hpk = _node_update(h, aggs, W_sc[0], W_lo[0])
    aggs9 = []
    for i in range(_C):
        xj = _sc_gather(hpk, srcs[i], _G_CHW)
        m9 = _edge_msg_l2(eas[i], shs[i], xj, W1[1], b1[1], W2[1], b2[1],
                          W_tp[1], W_pg[1], W_lo[1], W_out)
        o = _sc_scatter9(m9, dsts[i])
        aggs9.extend([o[:_NP], o[_NP:]])
    return _final(h, aggs9, W_sc[1], W_out, sphere_Y, sphere_dirs)
